# baseline (speedup 1.0000x reference)
import functools

import jax
import jax.numpy as jnp
from jax import lax
from jax.experimental import pallas as pl
from jax.experimental.pallas import tpu as pltpu
from jax.experimental.pallas import tpu_sc as plsc

DIM = 128
B, S = 4, 8192
N_ROWS = B * S
CHUNK = 128
_info = plsc.get_sparse_core_info()
NC, NS = _info.num_cores, _info.num_subcores
NW = NC * NS
PER_W = N_ROWS // NW
W_PER_B = S // PER_W
N_CHUNKS = PER_W // CHUNK
N_JOBS = 2 * N_CHUNKS
DEPTH = 7
AHEAD = 4

_mesh = plsc.VectorSubcoreMesh(core_axis_name="c", subcore_axis_name="s")


@functools.partial(
    pl.kernel,
    mesh=_mesh,
    out_type=(
        jax.ShapeDtypeStruct((N_ROWS, DIM), jnp.float32),
        jax.ShapeDtypeStruct((N_ROWS, DIM), jnp.float32),
    ),
    scratch_types=(
        [pltpu.VMEM((PER_W,), jnp.int32),
         pltpu.VMEM((DEPTH, CHUNK, DIM), jnp.float32)]
        + [pltpu.SemaphoreType.DMA] * (2 * DEPTH)
    ),
)
def _gather_kernel(cos_hbm, sin_hbm, idx_hbm, out_cos, out_sin,
                   idx_v, bufs, *sems):
    gsem = sems[:DEPTH]
    ssem = sems[DEPTH:]
    wid = lax.axis_index("s") * NC + lax.axis_index("c")
    base = wid * PER_W
    batch = wid // W_PER_B
    soff = (wid % W_PER_B) * PER_W
    pltpu.sync_copy(idx_hbm.at[batch, pl.ds(soff, PER_W)], idx_v)

    tables = (cos_hbm, sin_hbm)
    outs = (out_cos, out_sin)
    g_copies = [None] * DEPTH
    s_copies = [None] * DEPTH

    def issue_gather(k):
        sl = k % DEPTH
        chunk, tbl = k >> 1, k & 1
        g_copies[sl] = pltpu.async_copy(
            tables[tbl].at[idx_v.at[pl.ds(chunk * CHUNK, CHUNK)]],
            bufs.at[sl], gsem[sl])

    for k in range(AHEAD):
        issue_gather(k)
    for k in range(N_JOBS):
        sl = k % DEPTH
        if k + AHEAD < N_JOBS:
            nsl = (k + AHEAD) % DEPTH
            if s_copies[nsl] is not None:
                s_copies[nsl].wait()
                s_copies[nsl] = None
            issue_gather(k + AHEAD)
        g_copies[sl].wait()
        chunk, tbl = k >> 1, k & 1
        s_copies[sl] = pltpu.async_copy(
            bufs.at[sl], outs[tbl].at[pl.ds(base + chunk * CHUNK, CHUNK)],
            ssem[sl])
    for sl in range(DEPTH):
        if s_copies[sl] is not None:
            s_copies[sl].wait()


def kernel(x, position_ids, cos_cached, sin_cached):
    out_cos, out_sin = _gather_kernel(cos_cached, sin_cached,
                                      position_ids.astype(jnp.int32))
    shape = (*position_ids.shape, DIM)
    return (out_cos.reshape(shape).astype(x.dtype),
            out_sin.reshape(shape).astype(x.dtype))


# speedup vs baseline: 1.0236x; 1.0236x over previous
import functools

import jax
import jax.numpy as jnp
from jax import lax
from jax.experimental import pallas as pl
from jax.experimental.pallas import tpu as pltpu
from jax.experimental.pallas import tpu_sc as plsc

DIM = 128
B, S = 4, 8192
N_ROWS = B * S
CHUNK = 128
_info = plsc.get_sparse_core_info()
NC, NS = _info.num_cores, _info.num_subcores
NW = NC * NS
PER_W = N_ROWS // NW
W_PER_B = S // PER_W
N_CHUNKS = PER_W // CHUNK
N_JOBS = 2 * N_CHUNKS
DEPTH = 7
AHEAD = 4

_mesh = plsc.VectorSubcoreMesh(core_axis_name="c", subcore_axis_name="s")


@functools.partial(
    pl.kernel,
    mesh=_mesh,
    out_type=(
        jax.ShapeDtypeStruct((N_ROWS, DIM), jnp.float32),
        jax.ShapeDtypeStruct((N_ROWS, DIM), jnp.float32),
    ),
    scratch_types=(
        [pltpu.VMEM((PER_W,), jnp.int32),
         pltpu.VMEM((DEPTH, CHUNK, DIM), jnp.float32)]
        + [pltpu.SemaphoreType.DMA] * (2 * DEPTH + 1)
    ),
)
def _gather_kernel(cos_hbm, sin_hbm, idx_hbm, out_cos, out_sin,
                   idx_v, bufs, *sems):
    gsem = sems[:DEPTH]
    ssem = sems[DEPTH:2 * DEPTH]
    isem = sems[2 * DEPTH]
    wid = lax.axis_index("s") * NC + lax.axis_index("c")
    base = wid * PER_W
    batch = wid // W_PER_B
    soff = (wid % W_PER_B) * PER_W
    pltpu.sync_copy(idx_hbm.at[batch, pl.ds(soff, CHUNK)],
                    idx_v.at[pl.ds(0, CHUNK)])
    rest = pltpu.async_copy(
        idx_hbm.at[batch, pl.ds(soff + CHUNK, PER_W - CHUNK)],
        idx_v.at[pl.ds(CHUNK, PER_W - CHUNK)], isem)

    tables = (cos_hbm, sin_hbm)
    outs = (out_cos, out_sin)
    g_copies = [None] * DEPTH
    s_copies = [None] * DEPTH

    def issue_gather(k):
        sl = k % DEPTH
        chunk, tbl = k >> 1, k & 1
        g_copies[sl] = pltpu.async_copy(
            tables[tbl].at[idx_v.at[pl.ds(chunk * CHUNK, CHUNK)]],
            bufs.at[sl], gsem[sl])

    issue_gather(0)
    issue_gather(1)
    rest.wait()
    for k in range(2, AHEAD):
        issue_gather(k)
    for k in range(N_JOBS):
        sl = k % DEPTH
        if k + AHEAD < N_JOBS:
            nsl = (k + AHEAD) % DEPTH
            if s_copies[nsl] is not None:
                s_copies[nsl].wait()
                s_copies[nsl] = None
            issue_gather(k + AHEAD)
        g_copies[sl].wait()
        chunk, tbl = k >> 1, k & 1
        s_copies[sl] = pltpu.async_copy(
            bufs.at[sl], outs[tbl].at[pl.ds(base + chunk * CHUNK, CHUNK)],
            ssem[sl])
    for sl in range(DEPTH):
        if s_copies[sl] is not None:
            s_copies[sl].wait()


def kernel(x, position_ids, cos_cached, sin_cached):
    out_cos, out_sin = _gather_kernel(cos_cached, sin_cached,
                                      position_ids.astype(jnp.int32))
    shape = (*position_ids.shape, DIM)
    return (out_cos.reshape(shape).astype(x.dtype),
            out_sin.reshape(shape).astype(x.dtype))
